# baseline (speedup 1.0000x reference)
import jax
import jax.numpy as jnp
from jax import lax
from jax.experimental import pallas as pl
from jax.experimental.pallas import tpu as pltpu
from jax.experimental.pallas import tpu_sc as plsc

N = 320000
D = 128
NV = D // 16
B = 10000
NW = 32
SEG_BASE = 312
SEG_MAX = 328
RB = 128
CH = 16
NEGF = -3.0e38


def _sc_body(x_hbm, ids_hbm, bounds_hbm, out_hbm,
             bvec, xbuf0, xbuf1, idbuf0, idbuf1,
             sumacc, maxacc, cntbuf, csbuf, stage, sem0, sem1):
    wid = lax.axis_index("s") * 2 + lax.axis_index("c")
    seg_lo = wid * SEG_BASE
    is_last = (wid == NW - 1).astype(jnp.int32)
    nchunks = SEG_BASE // 8 + is_last * ((SEG_MAX - SEG_BASE) // 8)

    pltpu.sync_copy(bounds_hbm.at[wid], bvec)
    bv = bvec[...]
    lane = lax.broadcasted_iota(jnp.int32, (16,), 0)
    one_hot0 = 1 - jnp.minimum(lane, 1)
    rs = bv[0]
    re = bv[1]

    zero16 = jnp.zeros((16,), jnp.float32)
    ninf16 = jnp.full((16,), NEGF, jnp.float32)
    zcnt = jnp.zeros((16,), jnp.int32)

    def init_body(i, _):
        for j in range(NV):
            sumacc[i, pl.ds(16 * j, 16)] = zero16
            maxacc[i, pl.ds(16 * j, 16)] = ninf16
        return 0
    lax.fori_loop(0, SEG_MAX, init_body, 0)

    def cinit_body(i, _):
        cntbuf[pl.ds(16 * i, 16)] = zcnt
        return 0
    lax.fori_loop(0, (SEG_MAX + 16) // 16, cinit_body, 0)

    def flush(lid_c, cnt_c, cums, maxs):
        for j in range(NV):
            sl = pl.ds(16 * j, 16)
            plsc.addupdate(sumacc.at[lid_c, sl], cums[j] - csbuf[sl])
            m_old = maxacc[lid_c, sl]
            maxacc[lid_c, sl] = jnp.maximum(m_old, maxs[j])
        plsc.addupdate(cntbuf.at[pl.ds(lid_c, 16)], one_hot0 * cnt_c)

    def chunk_work(xb, ib, cb, lo_r, hi_r, masked):
        idv = ib[pl.ds(cb, 16)]
        lid_c = jnp.int32(-1)
        cnt_c = jnp.int32(0)
        cums = [zero16] * NV
        maxs = [ninf16] * NV
        for rr in range(CH):
            r = cb + rr
            nlid = idv[rr] - seg_lo
            xs = [xb[r, pl.ds(16 * j, 16)] for j in range(NV)]
            if masked:
                val_i = ((r >= lo_r) & (r < hi_r)).astype(jnp.int32)
                ch = (nlid != lid_c).astype(jnp.int32) * val_i
            else:
                val_i = jnp.int32(1)
                ch = (nlid != lid_c).astype(jnp.int32)
            ch_b = ch > 0

            @pl.when(ch_b)
            def _(lid_c=lid_c, cnt_c=cnt_c, cums=cums, maxs=maxs):
                @pl.when(lid_c >= 0)
                def _():
                    flush(lid_c, cnt_c, cums, maxs)
                for j in range(NV):
                    csbuf[pl.ds(16 * j, 16)] = cums[j]

            ch_f = ch.astype(jnp.float32)
            pen = jnp.full((16,), ch_f * NEGF, jnp.float32)
            if masked:
                val_f = val_i.astype(jnp.float32)
                vgate = jnp.full((16,), val_f, jnp.float32)
                vpen = jnp.full((16,), (1.0 - val_f) * NEGF, jnp.float32)
                cums = [cums[j] + xs[j] * vgate for j in range(NV)]
                maxs = [jnp.maximum(maxs[j] + pen, xs[j] * vgate + vpen)
                        for j in range(NV)]
                lid_c = nlid * val_i + lid_c * (1 - val_i)
            else:
                cums = [cums[j] + xs[j] for j in range(NV)]
                maxs = [jnp.maximum(maxs[j] + pen, xs[j])
                        for j in range(NV)]
                lid_c = nlid
            cnt_c = cnt_c * (1 - ch) + val_i

        @pl.when(lid_c >= 0)
        def _():
            flush(lid_c, cnt_c, cums, maxs)

    def start_dma(kb, xb, ib, sem):
        base = kb * RB
        pltpu.async_copy(x_hbm.at[pl.ds(base, RB)], xb, sem)
        pltpu.async_copy(ids_hbm.at[pl.ds(base, RB)], ib.at[pl.ds(0, RB)], sem)

    def wait_dma(kb, xb, ib, sem):
        base = kb * RB
        pltpu.make_async_copy(x_hbm.at[pl.ds(base, RB)], xb, sem).wait()
        pltpu.make_async_copy(ids_hbm.at[pl.ds(base, RB)],
                              ib.at[pl.ds(0, RB)], sem).wait()

    def process(kb, xb, ib):
        base = kb * RB
        lo_r = jnp.maximum(rs - base, 0)
        hi_r = jnp.minimum(re - base, RB)

        def chunk_body(c, _):
            cb = c * CH
            full = jnp.logical_and(cb >= lo_r, cb + CH <= hi_r)

            @pl.when(full)
            def _():
                chunk_work(xb, ib, cb, lo_r, hi_r, False)

            @pl.when(jnp.logical_not(full))
            def _():
                chunk_work(xb, ib, cb, lo_r, hi_r, True)
            return 0

        lax.fori_loop(lo_r // CH, (hi_r + CH - 1) // CH, chunk_body, 0)

    kb_lo = rs // RB
    kb_hi = (re + RB - 1) // RB

    @pl.when(kb_lo < kb_hi)
    def _prologue():
        start_dma(kb_lo, xbuf0, idbuf0, sem0)

    def pair_body(p, _):
        b0 = kb_lo + 2 * p
        b1 = b0 + 1
        wait_dma(b0, xbuf0, idbuf0, sem0)

        @pl.when(b1 < kb_hi)
        def _():
            start_dma(b1, xbuf1, idbuf1, sem1)
        process(b0, xbuf0, idbuf0)

        @pl.when(b1 < kb_hi)
        def _():
            wait_dma(b1, xbuf1, idbuf1, sem1)

            @pl.when(b1 + 1 < kb_hi)
            def _():
                start_dma(b1 + 1, xbuf0, idbuf0, sem0)
            process(b1, xbuf1, idbuf1)
        return 0

    npairs = (kb_hi - kb_lo + 1) // 2
    lax.fori_loop(0, npairs, pair_body, 0)

    def fin_body(cidx, _):
        cload = cntbuf[pl.ds(cidx * 8, 16)]
        for s in range(8):
            row = cidx * 8 + s
            cs = cload[s]
            denom = jnp.maximum(jnp.full((16,), cs, jnp.int32),
                                1).astype(jnp.float32)
            for j in range(NV):
                sl = pl.ds(16 * j, 16)
                sv = sumacc[row, sl]
                stage[s, sl] = sv
                stage[s, pl.ds(D + 16 * j, 16)] = sv / denom
                stage[s, pl.ds(2 * D + 16 * j, 16)] = maxacc[row, sl]

            @pl.when(cs == 0)
            def _zero_row():
                for j in range(3 * NV):
                    stage[s, pl.ds(16 * j, 16)] = zero16
        pltpu.sync_copy(stage, out_hbm.at[pl.ds(seg_lo + cidx * 8, 8)])
        return 0
    lax.fori_loop(0, nchunks, fin_body, 0)


@jax.jit
def _run(x, batch_i32, bounds):
    mesh = plsc.VectorSubcoreMesh(core_axis_name="c", subcore_axis_name="s")
    f = pl.kernel(
        _sc_body,
        out_type=jax.ShapeDtypeStruct((B, 3 * D), jnp.float32),
        mesh=mesh,
        scratch_types=[
            pltpu.VMEM((16,), jnp.int32),
            pltpu.VMEM((RB, D), jnp.float32),
            pltpu.VMEM((RB, D), jnp.float32),
            pltpu.VMEM((RB + 16,), jnp.int32),
            pltpu.VMEM((RB + 16,), jnp.int32),
            pltpu.VMEM((SEG_MAX, D), jnp.float32),
            pltpu.VMEM((SEG_MAX, D), jnp.float32),
            pltpu.VMEM((SEG_MAX + 16,), jnp.int32),
            pltpu.VMEM((D,), jnp.float32),
            pltpu.VMEM((8, 3 * D), jnp.float32),
            pltpu.SemaphoreType.DMA,
            pltpu.SemaphoreType.DMA,
        ],
    )
    return f(x, batch_i32, bounds)


def kernel(x, batch, batch_size):
    ids = batch.astype(jnp.int32)
    bvals = jnp.array([SEG_BASE * w for w in range(NW)] + [B], jnp.int32)
    bnds = jnp.searchsorted(ids, bvals, side="left").astype(jnp.int32)
    bounds = jnp.zeros((NW, 16), jnp.int32)
    bounds = bounds.at[:, 0].set(bnds[:NW]).at[:, 1].set(bnds[1:])
    return _run(x, ids, bounds)


# speedup vs baseline: 2.0390x; 2.0390x over previous
import jax
import jax.numpy as jnp
from jax import lax
from jax.experimental import pallas as pl
from jax.experimental.pallas import tpu as pltpu
from jax.experimental.pallas import tpu_sc as plsc

N = 320000
D = 128
NV = D // 16
B = 10000
NW = 32
SEG_BASE = 312
SEG_MAX = 328
RB = 128
CH = 16
NEGF = -3.0e38


def _sc_body(x_hbm, ids_hbm, bounds_hbm, out_hbm,
             bvec, xbuf0, xbuf1, idbuf0, idbuf1,
             sumacc, maxacc, cntbuf, csbuf, stage, sem0, sem1):
    wid = lax.axis_index("s") * 2 + lax.axis_index("c")
    seg_lo = wid * SEG_BASE
    is_last = (wid == NW - 1).astype(jnp.int32)
    nchunks = SEG_BASE // 8 + is_last * ((SEG_MAX - SEG_BASE) // 8)

    pltpu.sync_copy(bounds_hbm.at[wid], bvec)
    bv = bvec[...]
    lane = lax.broadcasted_iota(jnp.int32, (16,), 0)
    one_hot0 = 1 - jnp.minimum(lane, 1)
    rs = bv[0]
    re = bv[1]

    zero16 = jnp.zeros((16,), jnp.float32)
    ninf16 = jnp.full((16,), NEGF, jnp.float32)
    zcnt = jnp.zeros((16,), jnp.int32)

    def init_body(i, _):
        for j in range(NV):
            sumacc[i, pl.ds(16 * j, 16)] = zero16
            maxacc[i, pl.ds(16 * j, 16)] = ninf16
        return 0
    lax.fori_loop(0, SEG_MAX, init_body, 0)

    def cinit_body(i, _):
        cntbuf[pl.ds(16 * i, 16)] = zcnt
        return 0
    lax.fori_loop(0, (SEG_MAX + 16) // 16, cinit_body, 0)

    def flush(lid_c, cnt_c, sums, maxs):
        for j in range(NV):
            sl = pl.ds(16 * j, 16)
            plsc.addupdate(sumacc.at[lid_c, sl], sums[j])
            m_old = maxacc[lid_c, sl]
            maxacc[lid_c, sl] = jnp.maximum(m_old, maxs[j])
        plsc.addupdate(cntbuf.at[pl.ds(lid_c, 16)], one_hot0 * cnt_c)

    def chunk_work(xb, ib, cb, lo_r, hi_r, masked):
        idv = ib[pl.ds(cb, 16)]
        lid_c = jnp.int32(-1)
        cnt_c = jnp.int32(0)
        sums = [zero16] * NV
        maxs = [ninf16] * NV
        for rr in range(CH):
            r = cb + rr
            nlid = idv[rr] - seg_lo
            xs = [xb[r, pl.ds(16 * j, 16)] for j in range(NV)]
            if masked:
                val_i = ((r >= lo_r) & (r < hi_r)).astype(jnp.int32)
                ch = (nlid != lid_c).astype(jnp.int32) * val_i
            else:
                val_i = jnp.int32(1)
                ch = (nlid != lid_c).astype(jnp.int32)
            ch_b = ch > 0

            @pl.when(jnp.logical_and(ch_b, lid_c >= 0))
            def _(lid_c=lid_c, cnt_c=cnt_c, sums=sums, maxs=maxs):
                flush(lid_c, cnt_c, sums, maxs)

            ch_f = ch.astype(jnp.float32)
            alive = jnp.full((16,), 1.0 - ch_f, jnp.float32)
            pen = jnp.full((16,), ch_f * NEGF, jnp.float32)
            if masked:
                val_f = val_i.astype(jnp.float32)
                vgate = jnp.full((16,), val_f, jnp.float32)
                vpen = jnp.full((16,), (1.0 - val_f) * NEGF, jnp.float32)
                sums = [sums[j] * alive + xs[j] * vgate for j in range(NV)]
                maxs = [jnp.maximum(maxs[j] + pen, xs[j] * vgate + vpen)
                        for j in range(NV)]
                lid_c = nlid * val_i + lid_c * (1 - val_i)
            else:
                sums = [sums[j] * alive + xs[j] for j in range(NV)]
                maxs = [jnp.maximum(maxs[j] + pen, xs[j])
                        for j in range(NV)]
                lid_c = nlid
            cnt_c = cnt_c * (1 - ch) + val_i

        @pl.when(lid_c >= 0)
        def _():
            flush(lid_c, cnt_c, sums, maxs)

    def start_dma(kb, xb, ib, sem):
        base = kb * RB
        pltpu.async_copy(x_hbm.at[pl.ds(base, RB)], xb, sem)
        pltpu.async_copy(ids_hbm.at[pl.ds(base, RB)], ib.at[pl.ds(0, RB)], sem)

    def wait_dma(kb, xb, ib, sem):
        base = kb * RB
        pltpu.make_async_copy(x_hbm.at[pl.ds(base, RB)], xb, sem).wait()
        pltpu.make_async_copy(ids_hbm.at[pl.ds(base, RB)],
                              ib.at[pl.ds(0, RB)], sem).wait()

    def process(kb, xb, ib):
        base = kb * RB
        lo_r = jnp.maximum(rs - base, 0)
        hi_r = jnp.minimum(re - base, RB)

        def chunk_body(c, _):
            cb = c * CH
            full = jnp.logical_and(cb >= lo_r, cb + CH <= hi_r)

            @pl.when(full)
            def _():
                chunk_work(xb, ib, cb, lo_r, hi_r, False)

            @pl.when(jnp.logical_not(full))
            def _():
                chunk_work(xb, ib, cb, lo_r, hi_r, True)
            return 0

        lax.fori_loop(lo_r // CH, (hi_r + CH - 1) // CH, chunk_body, 0)

    kb_lo = rs // RB
    kb_hi = (re + RB - 1) // RB

    @pl.when(kb_lo < kb_hi)
    def _prologue():
        start_dma(kb_lo, xbuf0, idbuf0, sem0)

    def pair_body(p, _):
        b0 = kb_lo + 2 * p
        b1 = b0 + 1
        wait_dma(b0, xbuf0, idbuf0, sem0)

        @pl.when(b1 < kb_hi)
        def _():
            start_dma(b1, xbuf1, idbuf1, sem1)
        process(b0, xbuf0, idbuf0)

        @pl.when(b1 < kb_hi)
        def _():
            wait_dma(b1, xbuf1, idbuf1, sem1)

            @pl.when(b1 + 1 < kb_hi)
            def _():
                start_dma(b1 + 1, xbuf0, idbuf0, sem0)
            process(b1, xbuf1, idbuf1)
        return 0

    npairs = (kb_hi - kb_lo + 1) // 2
    lax.fori_loop(0, npairs, pair_body, 0)

    def fin_body(cidx, _):
        cload = cntbuf[pl.ds(cidx * 8, 16)]
        for s in range(8):
            row = cidx * 8 + s
            cs = cload[s]
            denom = jnp.maximum(jnp.full((16,), cs, jnp.int32),
                                1).astype(jnp.float32)
            for j in range(NV):
                sl = pl.ds(16 * j, 16)
                sv = sumacc[row, sl]
                stage[s, sl] = sv
                stage[s, pl.ds(D + 16 * j, 16)] = sv / denom
                stage[s, pl.ds(2 * D + 16 * j, 16)] = maxacc[row, sl]

            @pl.when(cs == 0)
            def _zero_row():
                for j in range(3 * NV):
                    stage[s, pl.ds(16 * j, 16)] = zero16
        pltpu.sync_copy(stage, out_hbm.at[pl.ds(seg_lo + cidx * 8, 8)])
        return 0
    lax.fori_loop(0, nchunks, fin_body, 0)


@jax.jit
def _run(x, batch_i32, bounds):
    mesh = plsc.VectorSubcoreMesh(core_axis_name="c", subcore_axis_name="s")
    f = pl.kernel(
        _sc_body,
        out_type=jax.ShapeDtypeStruct((B, 3 * D), jnp.float32),
        mesh=mesh,
        scratch_types=[
            pltpu.VMEM((16,), jnp.int32),
            pltpu.VMEM((RB, D), jnp.float32),
            pltpu.VMEM((RB, D), jnp.float32),
            pltpu.VMEM((RB + 16,), jnp.int32),
            pltpu.VMEM((RB + 16,), jnp.int32),
            pltpu.VMEM((SEG_MAX, D), jnp.float32),
            pltpu.VMEM((SEG_MAX, D), jnp.float32),
            pltpu.VMEM((SEG_MAX + 16,), jnp.int32),
            pltpu.VMEM((D,), jnp.float32),
            pltpu.VMEM((8, 3 * D), jnp.float32),
            pltpu.SemaphoreType.DMA,
            pltpu.SemaphoreType.DMA,
        ],
    )
    return f(x, batch_i32, bounds)


def kernel(x, batch, batch_size):
    ids = batch.astype(jnp.int32)
    bvals = jnp.array([SEG_BASE * w for w in range(NW)] + [B], jnp.int32)
    bnds = jnp.searchsorted(ids, bvals, side="left").astype(jnp.int32)
    bounds = jnp.zeros((NW, 16), jnp.int32)
    bounds = bounds.at[:, 0].set(bnds[:NW]).at[:, 1].set(bnds[1:])
    return _run(x, ids, bounds)


# speedup vs baseline: 2.0396x; 1.0003x over previous
import jax
import jax.numpy as jnp
from jax import lax
from jax.experimental import pallas as pl
from jax.experimental.pallas import tpu as pltpu
from jax.experimental.pallas import tpu_sc as plsc

N = 320000
D = 128
NV = D // 16
B = 10000
NW = 32
SEG_BASE = 312
SEG_MAX = 328
RB = 128
CH = 16
NEGF = -3.0e38


def _sc_body(x_hbm, ids_hbm, bounds_hbm, out_hbm,
             bvec, xbuf0, xbuf1, idbuf0, idbuf1,
             sumacc, maxacc, cntbuf, csbuf, stage, sem0, sem1):
    wid = lax.axis_index("s") * 2 + lax.axis_index("c")
    seg_lo = wid * SEG_BASE
    is_last = (wid == NW - 1).astype(jnp.int32)
    nchunks = SEG_BASE // 8 + is_last * ((SEG_MAX - SEG_BASE) // 8)

    pltpu.sync_copy(bounds_hbm.at[wid], bvec)
    bv = bvec[...]
    lane = lax.broadcasted_iota(jnp.int32, (16,), 0)
    one_hot0 = 1 - jnp.minimum(lane, 1)
    rs = bv[0]
    re = bv[1]

    zero16 = jnp.zeros((16,), jnp.float32)
    ninf16 = jnp.full((16,), NEGF, jnp.float32)
    zcnt = jnp.zeros((16,), jnp.int32)

    def init_body(i, _):
        for j in range(NV):
            sumacc[i, pl.ds(16 * j, 16)] = zero16
            maxacc[i, pl.ds(16 * j, 16)] = ninf16
        return 0
    lax.fori_loop(0, SEG_MAX, init_body, 0)

    def cinit_body(i, _):
        cntbuf[pl.ds(16 * i, 16)] = zcnt
        return 0
    lax.fori_loop(0, (SEG_MAX + 16) // 16, cinit_body, 0)

    def flush(lid_c, cnt_c, sums, maxs):
        for j in range(NV):
            sl = pl.ds(16 * j, 16)
            plsc.addupdate(sumacc.at[lid_c, sl], sums[j])
            m_old = maxacc[lid_c, sl]
            maxacc[lid_c, sl] = jnp.maximum(m_old, maxs[j])
        plsc.addupdate(cntbuf.at[pl.ds(lid_c, 16)], one_hot0 * cnt_c)

    def chunk_work(xb, ib, cb, lo_r, hi_r, masked):
        idv = ib[pl.ds(cb, 16)]
        lid_c = jnp.int32(-1)
        cnt_c = jnp.int32(0)
        sums = [zero16] * NV
        maxs = [ninf16] * NV
        for rr in range(CH):
            r = cb + rr
            nlid = idv[rr] - seg_lo
            xs = [xb[r, pl.ds(16 * j, 16)] for j in range(NV)]
            if masked:
                val_i = ((r >= lo_r) & (r < hi_r)).astype(jnp.int32)
                ch = (nlid != lid_c).astype(jnp.int32) * val_i
            else:
                val_i = jnp.int32(1)
                ch = (nlid != lid_c).astype(jnp.int32)
            do_flush = ch * (lid_c >= 0).astype(jnp.int32)

            def _fb(i, _, lid_c=lid_c, cnt_c=cnt_c, sums=sums, maxs=maxs):
                flush(lid_c, cnt_c, sums, maxs)
                return 0
            lax.fori_loop(0, do_flush, _fb, 0)

            ch_f = ch.astype(jnp.float32)
            alive = jnp.full((16,), 1.0 - ch_f, jnp.float32)
            pen = jnp.full((16,), ch_f * NEGF, jnp.float32)
            if masked:
                val_f = val_i.astype(jnp.float32)
                vgate = jnp.full((16,), val_f, jnp.float32)
                vpen = jnp.full((16,), (1.0 - val_f) * NEGF, jnp.float32)
                sums = [sums[j] * alive + xs[j] * vgate for j in range(NV)]
                maxs = [jnp.maximum(maxs[j] + pen, xs[j] * vgate + vpen)
                        for j in range(NV)]
                lid_c = nlid * val_i + lid_c * (1 - val_i)
            else:
                sums = [sums[j] * alive + xs[j] for j in range(NV)]
                maxs = [jnp.maximum(maxs[j] + pen, xs[j])
                        for j in range(NV)]
                lid_c = nlid
            cnt_c = cnt_c * (1 - ch) + val_i

        def _fb(i, _):
            flush(lid_c, cnt_c, sums, maxs)
            return 0
        lax.fori_loop(0, (lid_c >= 0).astype(jnp.int32), _fb, 0)

    def start_dma(kb, xb, ib, sem):
        base = kb * RB
        pltpu.async_copy(x_hbm.at[pl.ds(base, RB)], xb, sem)
        pltpu.async_copy(ids_hbm.at[pl.ds(base, RB)], ib.at[pl.ds(0, RB)], sem)

    def wait_dma(kb, xb, ib, sem):
        base = kb * RB
        pltpu.make_async_copy(x_hbm.at[pl.ds(base, RB)], xb, sem).wait()
        pltpu.make_async_copy(ids_hbm.at[pl.ds(base, RB)],
                              ib.at[pl.ds(0, RB)], sem).wait()

    def process(kb, xb, ib):
        base = kb * RB
        lo_r = jnp.maximum(rs - base, 0)
        hi_r = jnp.minimum(re - base, RB)

        def chunk_body(c, _):
            cb = c * CH
            full = jnp.logical_and(cb >= lo_r, cb + CH <= hi_r)

            @pl.when(full)
            def _():
                chunk_work(xb, ib, cb, lo_r, hi_r, False)

            @pl.when(jnp.logical_not(full))
            def _():
                chunk_work(xb, ib, cb, lo_r, hi_r, True)
            return 0

        lax.fori_loop(lo_r // CH, (hi_r + CH - 1) // CH, chunk_body, 0)

    kb_lo = rs // RB
    kb_hi = (re + RB - 1) // RB

    @pl.when(kb_lo < kb_hi)
    def _prologue():
        start_dma(kb_lo, xbuf0, idbuf0, sem0)

    def pair_body(p, _):
        b0 = kb_lo + 2 * p
        b1 = b0 + 1
        wait_dma(b0, xbuf0, idbuf0, sem0)

        @pl.when(b1 < kb_hi)
        def _():
            start_dma(b1, xbuf1, idbuf1, sem1)
        process(b0, xbuf0, idbuf0)

        @pl.when(b1 < kb_hi)
        def _():
            wait_dma(b1, xbuf1, idbuf1, sem1)

            @pl.when(b1 + 1 < kb_hi)
            def _():
                start_dma(b1 + 1, xbuf0, idbuf0, sem0)
            process(b1, xbuf1, idbuf1)
        return 0

    npairs = (kb_hi - kb_lo + 1) // 2
    lax.fori_loop(0, npairs, pair_body, 0)

    def fin_body(cidx, _):
        cload = cntbuf[pl.ds(cidx * 8, 16)]
        for s in range(8):
            row = cidx * 8 + s
            cs = cload[s]
            denom = jnp.maximum(jnp.full((16,), cs, jnp.int32),
                                1).astype(jnp.float32)
            for j in range(NV):
                sl = pl.ds(16 * j, 16)
                sv = sumacc[row, sl]
                stage[s, sl] = sv
                stage[s, pl.ds(D + 16 * j, 16)] = sv / denom
                stage[s, pl.ds(2 * D + 16 * j, 16)] = maxacc[row, sl]

            @pl.when(cs == 0)
            def _zero_row():
                for j in range(3 * NV):
                    stage[s, pl.ds(16 * j, 16)] = zero16
        pltpu.sync_copy(stage, out_hbm.at[pl.ds(seg_lo + cidx * 8, 8)])
        return 0
    lax.fori_loop(0, nchunks, fin_body, 0)


@jax.jit
def _run(x, batch_i32, bounds):
    mesh = plsc.VectorSubcoreMesh(core_axis_name="c", subcore_axis_name="s")
    f = pl.kernel(
        _sc_body,
        out_type=jax.ShapeDtypeStruct((B, 3 * D), jnp.float32),
        mesh=mesh,
        scratch_types=[
            pltpu.VMEM((16,), jnp.int32),
            pltpu.VMEM((RB, D), jnp.float32),
            pltpu.VMEM((RB, D), jnp.float32),
            pltpu.VMEM((RB + 16,), jnp.int32),
            pltpu.VMEM((RB + 16,), jnp.int32),
            pltpu.VMEM((SEG_MAX, D), jnp.float32),
            pltpu.VMEM((SEG_MAX, D), jnp.float32),
            pltpu.VMEM((SEG_MAX + 16,), jnp.int32),
            pltpu.VMEM((D,), jnp.float32),
            pltpu.VMEM((8, 3 * D), jnp.float32),
            pltpu.SemaphoreType.DMA,
            pltpu.SemaphoreType.DMA,
        ],
    )
    return f(x, batch_i32, bounds)


def kernel(x, batch, batch_size):
    ids = batch.astype(jnp.int32)
    bvals = jnp.array([SEG_BASE * w for w in range(NW)] + [B], jnp.int32)
    bnds = jnp.searchsorted(ids, bvals, side="left").astype(jnp.int32)
    bounds = jnp.zeros((NW, 16), jnp.int32)
    bounds = bounds.at[:, 0].set(bnds[:NW]).at[:, 1].set(bnds[1:])
    return _run(x, ids, bounds)


# speedup vs baseline: 2.9793x; 1.4608x over previous
import jax
import jax.numpy as jnp
from jax import lax
from jax.experimental import pallas as pl
from jax.experimental.pallas import tpu as pltpu
from jax.experimental.pallas import tpu_sc as plsc

N = 320000
D = 128
NV = D // 16
B = 10000
NW = 32
SEG_BASE = 312
SEG_MAX = 328
RB = 128
CH = 16
NEGF = -3.0e38


def _sc_body(x_hbm, ids_hbm, bounds_hbm, out_hbm,
             bvec, xbuf0, xbuf1, idbuf0, idbuf1,
             sumacc, maxacc, cntbuf, csbuf, stage, sem0, sem1):
    wid = lax.axis_index("s") * 2 + lax.axis_index("c")
    seg_lo = wid * SEG_BASE
    is_last = (wid == NW - 1).astype(jnp.int32)
    nchunks = SEG_BASE // 8 + is_last * ((SEG_MAX - SEG_BASE) // 8)

    pltpu.sync_copy(bounds_hbm.at[wid], bvec)
    bv = bvec[...]
    lane = lax.broadcasted_iota(jnp.int32, (16,), 0)
    one_hot0 = 1 - jnp.minimum(lane, 1)
    rs = bv[0]
    re = bv[1]

    zero16 = jnp.zeros((16,), jnp.float32)
    ninf16 = jnp.full((16,), NEGF, jnp.float32)
    zcnt = jnp.zeros((16,), jnp.int32)

    def init_body(i, _):
        for j in range(NV):
            sumacc[i, pl.ds(16 * j, 16)] = zero16
            maxacc[i, pl.ds(16 * j, 16)] = ninf16
        return 0
    lax.fori_loop(0, SEG_MAX, init_body, 0)

    def cinit_body(i, _):
        cntbuf[pl.ds(16 * i, 16)] = zcnt
        return 0
    lax.fori_loop(0, (SEG_MAX + 16) // 16, cinit_body, 0)

    def flush(lid_c, cnt_c, sums, maxs):
        for j in range(NV):
            sl = pl.ds(16 * j, 16)
            plsc.addupdate(sumacc.at[lid_c, sl], sums[j])
            m_old = maxacc[lid_c, sl]
            maxacc[lid_c, sl] = jnp.maximum(m_old, maxs[j])
        plsc.addupdate(cntbuf.at[pl.ds(lid_c, 16)], one_hot0 * cnt_c)

    def run_body(xb, ib, p, hi_r):
        idvec = ib[pl.ds(p, 16)]
        sid = idvec[0]
        lid = sid - seg_lo
        sums = [zero16] * NV
        maxs = [ninf16] * NV
        g = jnp.int32(1)
        run = jnp.int32(0)
        for rr in range(CH):
            if rr:
                g = g * (idvec[rr] == sid).astype(jnp.int32)
            gate = g * (p + rr < hi_r).astype(jnp.int32)
            run = run + gate
            gf = gate.astype(jnp.float32)
            gv = jnp.full((16,), gf, jnp.float32)
            gp = jnp.full((16,), (1.0 - gf) * NEGF, jnp.float32)
            xs = [xb[p + rr, pl.ds(16 * j, 16)] for j in range(NV)]
            xg = [xs[j] * gv for j in range(NV)]
            sums = [sums[j] + xg[j] for j in range(NV)]
            maxs = [jnp.maximum(maxs[j], xg[j] + gp) for j in range(NV)]
        flush(lid, run, sums, maxs)
        return p + run

    def start_dma(kb, xb, ib, sem):
        base = kb * RB
        pltpu.async_copy(x_hbm.at[pl.ds(base, RB)], xb.at[pl.ds(0, RB)], sem)
        pltpu.async_copy(ids_hbm.at[pl.ds(base, RB)], ib.at[pl.ds(0, RB)], sem)

    def wait_dma(kb, xb, ib, sem):
        base = kb * RB
        pltpu.make_async_copy(x_hbm.at[pl.ds(base, RB)], xb.at[pl.ds(0, RB)], sem).wait()
        pltpu.make_async_copy(ids_hbm.at[pl.ds(base, RB)],
                              ib.at[pl.ds(0, RB)], sem).wait()

    def process(kb, xb, ib):
        base = kb * RB
        lo_r = jnp.maximum(rs - base, 0)
        hi_r = jnp.minimum(re - base, RB)

        acc0 = jnp.zeros((16,), jnp.int32)

        def cntw(w, acc):
            cur = ib[pl.ds(w * 16 + 1, 16)]
            prev = ib[pl.ds(w * 16, 16)]
            dp = cur - prev
            ind = jnp.minimum(dp * dp, 1)
            pos = lane + (w * 16 + 1)
            t1 = jnp.minimum(jnp.maximum(pos - lo_r, 0), 1)
            t2 = jnp.minimum(jnp.maximum(hi_r - 1 - pos, -1) + 1, 1)
            return acc + ind * t1 * t2

        acc = lax.fori_loop(0, RB // 16, cntw, acc0)
        nb = acc[0]
        for ll in range(1, 16):
            nb = nb + acc[ll]
        nruns = (hi_r > lo_r).astype(jnp.int32) * (nb + 1)

        def rb_body(i, p):
            return run_body(xb, ib, p, hi_r)
        lax.fori_loop(0, nruns, rb_body, lo_r)

    kb_lo = rs // RB
    kb_hi = (re + RB - 1) // RB

    @pl.when(kb_lo < kb_hi)
    def _prologue():
        start_dma(kb_lo, xbuf0, idbuf0, sem0)

    def pair_body(p, _):
        b0 = kb_lo + 2 * p
        b1 = b0 + 1
        wait_dma(b0, xbuf0, idbuf0, sem0)

        @pl.when(b1 < kb_hi)
        def _():
            start_dma(b1, xbuf1, idbuf1, sem1)
        process(b0, xbuf0, idbuf0)

        @pl.when(b1 < kb_hi)
        def _():
            wait_dma(b1, xbuf1, idbuf1, sem1)

            @pl.when(b1 + 1 < kb_hi)
            def _():
                start_dma(b1 + 1, xbuf0, idbuf0, sem0)
            process(b1, xbuf1, idbuf1)
        return 0

    npairs = (kb_hi - kb_lo + 1) // 2
    lax.fori_loop(0, npairs, pair_body, 0)

    def fin_body(cidx, _):
        cload = cntbuf[pl.ds(cidx * 8, 16)]
        for s in range(8):
            row = cidx * 8 + s
            cs = cload[s]
            denom = jnp.maximum(jnp.full((16,), cs, jnp.int32),
                                1).astype(jnp.float32)
            for j in range(NV):
                sl = pl.ds(16 * j, 16)
                sv = sumacc[row, sl]
                stage[s, sl] = sv
                stage[s, pl.ds(D + 16 * j, 16)] = sv / denom
                stage[s, pl.ds(2 * D + 16 * j, 16)] = maxacc[row, sl]

            @pl.when(cs == 0)
            def _zero_row():
                for j in range(3 * NV):
                    stage[s, pl.ds(16 * j, 16)] = zero16
        pltpu.sync_copy(stage, out_hbm.at[pl.ds(seg_lo + cidx * 8, 8)])
        return 0
    lax.fori_loop(0, nchunks, fin_body, 0)


@jax.jit
def _run(x, batch_i32, bounds):
    mesh = plsc.VectorSubcoreMesh(core_axis_name="c", subcore_axis_name="s")
    f = pl.kernel(
        _sc_body,
        out_type=jax.ShapeDtypeStruct((B, 3 * D), jnp.float32),
        mesh=mesh,
        scratch_types=[
            pltpu.VMEM((16,), jnp.int32),
            pltpu.VMEM((RB + CH, D), jnp.float32),
            pltpu.VMEM((RB + CH, D), jnp.float32),
            pltpu.VMEM((RB + 16,), jnp.int32),
            pltpu.VMEM((RB + 16,), jnp.int32),
            pltpu.VMEM((SEG_MAX, D), jnp.float32),
            pltpu.VMEM((SEG_MAX, D), jnp.float32),
            pltpu.VMEM((SEG_MAX + 16,), jnp.int32),
            pltpu.VMEM((D,), jnp.float32),
            pltpu.VMEM((8, 3 * D), jnp.float32),
            pltpu.SemaphoreType.DMA,
            pltpu.SemaphoreType.DMA,
        ],
    )
    return f(x, batch_i32, bounds)


def kernel(x, batch, batch_size):
    ids = batch.astype(jnp.int32)
    bvals = jnp.array([SEG_BASE * w for w in range(NW)] + [B], jnp.int32)
    bnds = jnp.searchsorted(ids, bvals, side="left").astype(jnp.int32)
    bounds = jnp.zeros((NW, 16), jnp.int32)
    bounds = bounds.at[:, 0].set(bnds[:NW]).at[:, 1].set(bnds[1:])
    return _run(x, ids, bounds)
